# Initial kernel scaffold; baseline (speedup 1.0000x reference)
import jax
import jax.numpy as jnp
from jax.experimental import pallas as pl


def kernel(t, z, Wc0, bc0, Wc1, bc1, Wc2, bc2, Wo, bo, Wg0, bg0, Wg1, bg1, Wg2, bg2):
    raise NotImplementedError("write your pallas kernel here")


# speedup vs baseline: 1.6941x; 1.6941x over previous
import functools

import jax
import jax.numpy as jnp
from jax.experimental import pallas as pl


def _celu(x):
    return jnp.where(x > 0, x, jnp.exp(jnp.minimum(x, 0.0)) - 1.0)


def _softplus(x):
    return jnp.maximum(x, 0.0) + jnp.log(1.0 + jnp.exp(-jnp.abs(x)))


def _body(z_ref, wc0, bc0, wc1, bc1, wc2, bc2, wo, bo,
          wg0, bg0, wg1, bg1, wg2, bg2, o_ref):
    x = z_ref[...]
    dot = functools.partial(jnp.dot, preferred_element_type=jnp.float32)
    h = _celu(dot(x, wc0[...]) + bc0[...])
    h = _celu(dot(h, wc1[...]) + bc1[...])
    v = _celu(dot(h, wc2[...]) + bc2[...])
    f = dot(v, wo[...]) + bo[...]
    g = _celu(dot(x, wg0[...]) + bg0[...])
    g = _celu(dot(g, wg1[...]) + bg1[...])
    g = _softplus(dot(g, wg2[...]) + bg2[...])
    o_ref[...] = f - g * x


def kernel(t, z, Wc0, bc0, Wc1, bc1, Wc2, bc2, Wo, bo,
           Wg0, bg0, Wg1, bg1, Wg2, bg2):
    S, _, dim_z = z.shape
    dim_h = Wc0.shape[0]
    zf = z[:, 0, :]

    ws = [Wc0.T, bc0[None, :], Wc1.T, bc1[None, :], Wc2.T, bc2[None, :],
          Wo[:, :dim_h].T, bo[None, :],
          Wg0.T, bg0[None, :], Wg1.T, bg1[None, :], Wg2.T, bg2[None, :]]

    BLOCK = 1024
    grid = (S // BLOCK,)

    def wspec(a):
        return pl.BlockSpec(a.shape, lambda i: (0, 0))

    out = pl.pallas_call(
        _body,
        grid=grid,
        in_specs=[pl.BlockSpec((BLOCK, dim_z), lambda i: (i, 0))]
                 + [wspec(a) for a in ws],
        out_specs=pl.BlockSpec((BLOCK, dim_z), lambda i: (i, 0)),
        out_shape=jax.ShapeDtypeStruct((S, dim_z), jnp.float32),
    )(zf, *ws)
    return out[:, None, :]
